# baseline (speedup 1.0000x reference)
import functools

import jax
import jax.numpy as jnp
from jax import lax
from jax.experimental import pallas as pl
from jax.experimental.pallas import tpu as pltpu


def _make_tc_gather(B, V, D):
    NQ = 8
    UNROLL = 8

    def body(ids_ref, table_ref, out_ref, *sems):
        def loop(t, carry):
            j0 = t * UNROLL
            for u in range(UNROLL):
                j = j0 + u
                i = ids_ref[j]
                pltpu.async_copy(
                    table_ref.at[pl.ds(i, 1)],
                    out_ref.at[pl.ds(j, 1)],
                    sems[u % NQ],
                )
            return carry

        lax.fori_loop(0, B // UNROLL, loop, 0)
        per_q = B // NQ
        for q in range(NQ):
            pltpu.make_async_copy(
                table_ref.at[pl.ds(0, per_q)],
                out_ref.at[pl.ds(q * per_q, per_q)],
                sems[q],
            ).wait()

    return pl.pallas_call(
        body,
        grid=(),
        in_specs=[
            pl.BlockSpec(memory_space=pltpu.SMEM),
            pl.BlockSpec(memory_space=pl.ANY),
        ],
        out_specs=pl.BlockSpec(memory_space=pl.ANY),
        out_shape=jax.ShapeDtypeStruct((B, D), jnp.float32),
        scratch_shapes=[pltpu.SemaphoreType.DMA] * NQ,
    )


def kernel(style_ids, table):
    (B,) = style_ids.shape
    V, D = table.shape
    return _make_tc_gather(B, V, D)(style_ids.astype(jnp.int32), table)
